# baseline (speedup 1.0000x reference)
import functools

import jax
import jax.numpy as jnp
from jax import lax
from jax.experimental import pallas as pl
from jax.experimental.pallas import tpu as pltpu

B = 1024
K = 100000
D = 128
KT = 512
KP = ((K + KT - 1) // KT) * KT
C = 128
S = 20
BT = 8


def _nrm(x):
    ss = jnp.sum(x * x, axis=1, keepdims=True)
    return x * lax.rsqrt(jnp.maximum(ss, jnp.float32(1e-24)))


def _scores_body(state_ref, mem_ref, out_ref, qn_ref):
    k = pl.program_id(0)

    @pl.when(k == 0)
    def _():
        qn_ref[...] = _nrm(state_ref[...]).astype(jnp.bfloat16)

    m = mem_ref[...]
    s = lax.dot_general(qn_ref[...], _nrm(m).astype(jnp.bfloat16),
                        (((1,), (1,)), ((), ())),
                        preferred_element_type=jnp.float32)
    col = k * KT + lax.broadcasted_iota(jnp.int32, (B, KT), 1)
    out_ref[...] = jnp.where(col >= K, jnp.float32(-2.0), s)


def _scores(state_past, memory_past):
    mem_p = jnp.pad(memory_past, ((0, KP - K), (0, 0)))
    return pl.pallas_call(
        _scores_body,
        grid=(KP // KT,),
        in_specs=[
            pl.BlockSpec((B, D), lambda k: (0, 0)),
            pl.BlockSpec((KT, D), lambda k: (k, 0)),
        ],
        out_specs=pl.BlockSpec((B, KT), lambda k: (0, k)),
        out_shape=jax.ShapeDtypeStruct((B, KP), jnp.float32),
        scratch_shapes=[pltpu.VMEM((B, D), jnp.bfloat16)],
    )(state_past, mem_p)


def _bdot(x, w):
    return lax.dot_general(x.astype(jnp.bfloat16), w.astype(jnp.bfloat16),
                           (((1,), (0,)), ((), ())),
                           preferred_element_type=jnp.float32)


def _selector_body(state_ref, mps_ref,
                   wq1, bq1, wq2, bq2, wq3, bq3,
                   wm1, bm1, wm2, bm2, wm3, bm3,
                   q_ref, mo_ref):
    x = state_ref[...]
    h = jnp.maximum(_bdot(x, wq1[...]) + bq1[...], 0.0)
    h = jnp.maximum(_bdot(h, wq2[...]) + bq2[...], 0.0)
    q_ref[...] = _bdot(h, wq3[...]) + bq3[...]

    m = mps_ref[...].reshape(BT * C, D)
    hm = jnp.maximum(_bdot(m, wm1[...]) + bm1[...], 0.0)
    hm = jnp.maximum(_bdot(hm, wm2[...]) + bm2[...], 0.0)
    mo_ref[...] = (_bdot(hm, wm3[...]) + bm3[...]).reshape(BT, C, D)


def _selector(state_past, mem_past_sel,
              Wq1, bq1, Wq2, bq2, Wq3, bq3,
              Wm1, bm1, Wm2, bm2, Wm3, bm3):
    w_specs = []
    w_args = []
    for w, b in ((Wq1, bq1), (Wq2, bq2), (Wq3, bq3),
                 (Wm1, bm1), (Wm2, bm2), (Wm3, bm3)):
        b2 = b.reshape(1, -1)
        w_specs.append(pl.BlockSpec(w.shape, lambda i: (0, 0)))
        w_specs.append(pl.BlockSpec(b2.shape, lambda i: (0, 0)))
        w_args.append(w)
        w_args.append(b2)
    return pl.pallas_call(
        _selector_body,
        grid=(B // BT,),
        in_specs=[
            pl.BlockSpec((BT, D), lambda i: (i, 0)),
            pl.BlockSpec((BT, C, D), lambda i: (i, 0, 0)),
        ] + w_specs,
        out_specs=[
            pl.BlockSpec((BT, D), lambda i: (i, 0)),
            pl.BlockSpec((BT, C, D), lambda i: (i, 0, 0)),
        ],
        out_shape=[
            jax.ShapeDtypeStruct((B, D), jnp.float32),
            jax.ShapeDtypeStruct((B, C, D), jnp.float32),
        ],
    )(state_past, mem_past_sel, *w_args)


def _l2n(x, axis):
    n = jnp.linalg.norm(x, axis=axis, keepdims=True)
    return x / jnp.clip(n, 1e-12, None)


def kernel(state_past, memory_past, memory_fut, Wq1, bq1, Wq2, bq2, Wq3, bq3,
           Wm1, bm1, Wm2, bm2, Wm3, bm3):
    weight_read = _scores(state_past, memory_past)[:, :K]
    _, index_max = jax.lax.top_k(weight_read, C)
    mem_past_sel = jnp.take(memory_past, index_max, axis=0)
    q, mo = _selector(state_past, mem_past_sel,
                      Wq1, bq1, Wq2, bq2, Wq3, bq3,
                      Wm1, bm1, Wm2, bm2, Wm3, bm3)
    w2 = jnp.einsum('bqd,bcd->bc', _l2n(q[:, None, :], 2), _l2n(mo, 2))
    _, smi = jax.lax.top_k(w2, S)
    idx_final = jnp.take_along_axis(index_max, smi, axis=1)
    return jnp.take(memory_fut, idx_final, axis=0)


# speedup vs baseline: 1.5222x; 1.5222x over previous
import functools

import jax
import jax.numpy as jnp
from jax import lax
from jax.experimental import pallas as pl
from jax.experimental.pallas import tpu as pltpu
from jax.experimental.pallas import tpu_sc as plsc

B = 1024
K = 100000
D = 128
KT = 512
KP = ((K + KT - 1) // KT) * KT
C = 128
S = 20
BT = 8


def _nrm(x):
    ss = jnp.sum(x * x, axis=1, keepdims=True)
    return x * lax.rsqrt(jnp.maximum(ss, jnp.float32(1e-24)))


def _scores_body(state_ref, mem_ref, out_ref, qn_ref):
    k = pl.program_id(0)

    @pl.when(k == 0)
    def _():
        qn_ref[...] = _nrm(state_ref[...]).astype(jnp.bfloat16)

    m = mem_ref[...]
    s = lax.dot_general(qn_ref[...], _nrm(m).astype(jnp.bfloat16),
                        (((1,), (1,)), ((), ())),
                        preferred_element_type=jnp.float32)
    col = k * KT + lax.broadcasted_iota(jnp.int32, (B, KT), 1)
    out_ref[...] = jnp.where(col >= K, jnp.float32(-2.0), s)


def _scores(state_past, memory_past):
    mem_p = jnp.pad(memory_past, ((0, KP - K), (0, 0)))
    return pl.pallas_call(
        _scores_body,
        grid=(KP // KT,),
        in_specs=[
            pl.BlockSpec((B, D), lambda k: (0, 0)),
            pl.BlockSpec((KT, D), lambda k: (k, 0)),
        ],
        out_specs=pl.BlockSpec((B, KT), lambda k: (0, k)),
        out_shape=jax.ShapeDtypeStruct((B, KP), jnp.float32),
        scratch_shapes=[pltpu.VMEM((B, D), jnp.bfloat16)],
    )(state_past, mem_p)


def _bdot(x, w):
    return lax.dot_general(x.astype(jnp.bfloat16), w.astype(jnp.bfloat16),
                           (((1,), (0,)), ((), ())),
                           preferred_element_type=jnp.float32)


def _selector_body(state_ref, mps_ref,
                   wq1, bq1, wq2, bq2, wq3, bq3,
                   wm1, bm1, wm2, bm2, wm3, bm3,
                   q_ref, mo_ref):
    x = state_ref[...]
    h = jnp.maximum(_bdot(x, wq1[...]) + bq1[...], 0.0)
    h = jnp.maximum(_bdot(h, wq2[...]) + bq2[...], 0.0)
    q_ref[...] = _bdot(h, wq3[...]) + bq3[...]

    m = mps_ref[...].reshape(BT * C, D)
    hm = jnp.maximum(_bdot(m, wm1[...]) + bm1[...], 0.0)
    hm = jnp.maximum(_bdot(hm, wm2[...]) + bm2[...], 0.0)
    mo_ref[...] = (_bdot(hm, wm3[...]) + bm3[...]).reshape(BT, C, D)


def _selector(state_past, mem_past_sel,
              Wq1, bq1, Wq2, bq2, Wq3, bq3,
              Wm1, bm1, Wm2, bm2, Wm3, bm3):
    w_specs = []
    w_args = []
    for w, b in ((Wq1, bq1), (Wq2, bq2), (Wq3, bq3),
                 (Wm1, bm1), (Wm2, bm2), (Wm3, bm3)):
        b2 = b.reshape(1, -1)
        w_specs.append(pl.BlockSpec(w.shape, lambda i: (0, 0)))
        w_specs.append(pl.BlockSpec(b2.shape, lambda i: (0, 0)))
        w_args.append(w)
        w_args.append(b2)
    return pl.pallas_call(
        _selector_body,
        grid=(B // BT,),
        in_specs=[
            pl.BlockSpec((BT, D), lambda i: (i, 0)),
            pl.BlockSpec((BT, C, D), lambda i: (i, 0, 0)),
        ] + w_specs,
        out_specs=[
            pl.BlockSpec((BT, D), lambda i: (i, 0)),
            pl.BlockSpec((BT, C, D), lambda i: (i, 0, 0)),
        ],
        out_shape=[
            jax.ShapeDtypeStruct((B, D), jnp.float32),
            jax.ShapeDtypeStruct((B, C, D), jnp.float32),
        ],
    )(state_past, mem_past_sel, *w_args)


def _l2n(x, axis):
    n = jnp.linalg.norm(x, axis=axis, keepdims=True)
    return x / jnp.clip(n, 1e-12, None)


def _sc_gather_fut(memory_fut, idx_flat):
    info = plsc.get_sparse_core_info()
    nw = info.num_cores * info.num_subcores
    n = idx_flat.shape[0]
    per_w = n // nw
    mesh = plsc.VectorSubcoreMesh(core_axis_name="c", subcore_axis_name="s")

    @functools.partial(
        pl.kernel, mesh=mesh,
        out_type=jax.ShapeDtypeStruct((n, D), jnp.float32),
        scratch_types=[
            pltpu.VMEM((per_w,), jnp.int32),
            pltpu.VMEM((per_w, D), jnp.float32),
            pltpu.SemaphoreType.DMA,
        ],
    )
    def k(table_hbm, idx_hbm, out_hbm, idx_v, rows_v, sem):
        wid = lax.axis_index("s") * info.num_cores + lax.axis_index("c")
        base = wid * per_w
        pltpu.sync_copy(idx_hbm.at[pl.ds(base, per_w)], idx_v)
        pltpu.async_copy(table_hbm.at[idx_v], rows_v, sem).wait()
        pltpu.sync_copy(rows_v, out_hbm.at[pl.ds(base, per_w)])

    return k(memory_fut, idx_flat)


def kernel(state_past, memory_past, memory_fut, Wq1, bq1, Wq2, bq2, Wq3, bq3,
           Wm1, bm1, Wm2, bm2, Wm3, bm3):
    SEG = 2048
    NSEG = KP // SEG
    scores = _scores(state_past, memory_past).reshape(B, NSEG, SEG)
    v1, p1 = jax.lax.top_k(scores, C)
    gidx = p1 + (jnp.arange(NSEG, dtype=jnp.int32) * SEG)[None, :, None]
    v2, p2 = jax.lax.top_k(v1.reshape(B, NSEG * C), C)
    index_max = jnp.take_along_axis(gidx.reshape(B, NSEG * C), p2, axis=1)
    mem_past_sel = jnp.take(memory_past, index_max, axis=0)
    q, mo = _selector(state_past, mem_past_sel,
                      Wq1, bq1, Wq2, bq2, Wq3, bq3,
                      Wm1, bm1, Wm2, bm2, Wm3, bm3)
    w2 = jnp.einsum('bqd,bcd->bc', _l2n(q[:, None, :], 2), _l2n(mo, 2))
    _, smi = jax.lax.top_k(w2, S)
    idx_final = jnp.take_along_axis(index_max, smi, axis=1)
    return _sc_gather_fut(memory_fut, idx_final.reshape(-1)).reshape(B, S, D)


# speedup vs baseline: 14.3711x; 9.4410x over previous
import functools

import jax
import jax.numpy as jnp
from jax import lax
from jax.experimental import pallas as pl
from jax.experimental.pallas import tpu as pltpu
from jax.experimental.pallas import tpu_sc as plsc

B = 1024
K = 100000
D = 128
KT = 512
KP = ((K + KT - 1) // KT) * KT
C = 128
S = 20
BT = 8


def _nrm(x):
    ss = jnp.sum(x * x, axis=1, keepdims=True)
    return x * lax.rsqrt(jnp.maximum(ss, jnp.float32(1e-24)))


SEG = 128
NSEG = KP // SEG


def _scores_body(state_ref, mem_ref, out_ref, segmax_ref, qn_ref):
    k = pl.program_id(0)

    @pl.when(k == 0)
    def _():
        qn_ref[...] = _nrm(state_ref[...]).astype(jnp.bfloat16)

    m = mem_ref[...]
    s = lax.dot_general(qn_ref[...], _nrm(m).astype(jnp.bfloat16),
                        (((1,), (1,)), ((), ())),
                        preferred_element_type=jnp.float32)
    col = k * KT + lax.broadcasted_iota(jnp.int32, (B, KT), 1)
    s = jnp.where(col >= K, jnp.float32(-2.0), s)
    out_ref[...] = s
    segmax_ref[...] = jnp.max(s.reshape(B, KT // SEG, SEG), axis=2)[None]


def _scores(state_past, memory_past):
    mem_p = jnp.pad(memory_past, ((0, KP - K), (0, 0)))
    return pl.pallas_call(
        _scores_body,
        grid=(KP // KT,),
        in_specs=[
            pl.BlockSpec((B, D), lambda k: (0, 0)),
            pl.BlockSpec((KT, D), lambda k: (k, 0)),
        ],
        out_specs=[
            pl.BlockSpec((B, KT), lambda k: (0, k)),
            pl.BlockSpec((1, B, KT // SEG), lambda k: (k, 0, 0)),
        ],
        out_shape=[
            jax.ShapeDtypeStruct((B, KP), jnp.float32),
            jax.ShapeDtypeStruct((KP // KT, B, KT // SEG), jnp.float32),
        ],
        scratch_shapes=[pltpu.VMEM((B, D), jnp.bfloat16)],
    )(state_past, mem_p)


def _bdot(x, w):
    return lax.dot_general(x.astype(jnp.bfloat16), w.astype(jnp.bfloat16),
                           (((1,), (0,)), ((), ())),
                           preferred_element_type=jnp.float32)


def _selector_body(state_ref, mps_ref,
                   wq1, bq1, wq2, bq2, wq3, bq3,
                   wm1, bm1, wm2, bm2, wm3, bm3,
                   q_ref, mo_ref):
    x = state_ref[...]
    h = jnp.maximum(_bdot(x, wq1[...]) + bq1[...], 0.0)
    h = jnp.maximum(_bdot(h, wq2[...]) + bq2[...], 0.0)
    q_ref[...] = _bdot(h, wq3[...]) + bq3[...]

    m = mps_ref[...].reshape(BT * C, D)
    hm = jnp.maximum(_bdot(m, wm1[...]) + bm1[...], 0.0)
    hm = jnp.maximum(_bdot(hm, wm2[...]) + bm2[...], 0.0)
    mo_ref[...] = (_bdot(hm, wm3[...]) + bm3[...]).reshape(BT, C, D)


def _selector(state_past, mem_past_sel,
              Wq1, bq1, Wq2, bq2, Wq3, bq3,
              Wm1, bm1, Wm2, bm2, Wm3, bm3):
    w_specs = []
    w_args = []
    for w, b in ((Wq1, bq1), (Wq2, bq2), (Wq3, bq3),
                 (Wm1, bm1), (Wm2, bm2), (Wm3, bm3)):
        b2 = b.reshape(1, -1)
        w_specs.append(pl.BlockSpec(w.shape, lambda i: (0, 0)))
        w_specs.append(pl.BlockSpec(b2.shape, lambda i: (0, 0)))
        w_args.append(w)
        w_args.append(b2)
    return pl.pallas_call(
        _selector_body,
        grid=(B // BT,),
        in_specs=[
            pl.BlockSpec((BT, D), lambda i: (i, 0)),
            pl.BlockSpec((BT, C, D), lambda i: (i, 0, 0)),
        ] + w_specs,
        out_specs=[
            pl.BlockSpec((BT, D), lambda i: (i, 0)),
            pl.BlockSpec((BT, C, D), lambda i: (i, 0, 0)),
        ],
        out_shape=[
            jax.ShapeDtypeStruct((B, D), jnp.float32),
            jax.ShapeDtypeStruct((B, C, D), jnp.float32),
        ],
    )(state_past, mem_past_sel, *w_args)


def _l2n(x, axis):
    n = jnp.linalg.norm(x, axis=axis, keepdims=True)
    return x / jnp.clip(n, 1e-12, None)


def _sc_gather_fut(memory_fut, idx_flat):
    info = plsc.get_sparse_core_info()
    nw = info.num_cores * info.num_subcores
    n = idx_flat.shape[0]
    per_w = n // nw
    mesh = plsc.VectorSubcoreMesh(core_axis_name="c", subcore_axis_name="s")

    @functools.partial(
        pl.kernel, mesh=mesh,
        out_type=jax.ShapeDtypeStruct((n, D), jnp.float32),
        scratch_types=[
            pltpu.VMEM((per_w,), jnp.int32),
            pltpu.VMEM((per_w, D), jnp.float32),
            pltpu.SemaphoreType.DMA,
        ],
    )
    def k(table_hbm, idx_hbm, out_hbm, idx_v, rows_v, sem):
        wid = lax.axis_index("s") * info.num_cores + lax.axis_index("c")
        base = wid * per_w
        pltpu.sync_copy(idx_hbm.at[pl.ds(base, per_w)], idx_v)
        pltpu.async_copy(table_hbm.at[idx_v], rows_v, sem).wait()
        pltpu.sync_copy(rows_v, out_hbm.at[pl.ds(base, per_w)])

    return k(memory_fut, idx_flat)


def kernel(state_past, memory_past, memory_fut, Wq1, bq1, Wq2, bq2, Wq3, bq3,
           Wm1, bm1, Wm2, bm2, Wm3, bm3):
    scores, segmax3 = _scores(state_past, memory_past)
    segmax = jnp.transpose(segmax3, (1, 0, 2)).reshape(B, NSEG)
    _, seg1 = jax.lax.top_k(segmax, C)
    g1 = jnp.sort(seg1, axis=1)
    s1 = jnp.take_along_axis(scores.reshape(B, NSEG, SEG), g1[:, :, None],
                             axis=1)
    s1f = s1.reshape(B, C * SEG)
    m2 = jnp.max(s1f.reshape(B, (C * SEG) // 16, 16), axis=2)
    _, ch2 = jax.lax.top_k(m2, C)
    g2 = jnp.sort(ch2, axis=1)
    s2 = jnp.take_along_axis(s1f.reshape(B, (C * SEG) // 16, 16),
                             g2[:, :, None], axis=1)
    _, p2 = jax.lax.top_k(s2.reshape(B, C * 16), C)
    pos1 = jnp.take_along_axis(g2, p2 // 16, axis=1) * 16 + p2 % 16
    index_max = (jnp.take_along_axis(g1, pos1 // SEG, axis=1) * SEG
                 + pos1 % SEG)
    mem_past_sel = jnp.take(memory_past, index_max, axis=0)
    q, mo = _selector(state_past, mem_past_sel,
                      Wq1, bq1, Wq2, bq2, Wq3, bq3,
                      Wm1, bm1, Wm2, bm2, Wm3, bm3)
    w2 = jnp.einsum('bqd,bcd->bc', _l2n(q[:, None, :], 2), _l2n(mo, 2))
    _, smi = jax.lax.top_k(w2, S)
    idx_final = jnp.take_along_axis(index_max, smi, axis=1)
    return _sc_gather_fut(memory_fut, idx_final.reshape(-1)).reshape(B, S, D)


# speedup vs baseline: 15.6908x; 1.0918x over previous
import functools

import jax
import jax.numpy as jnp
from jax import lax
from jax.experimental import pallas as pl
from jax.experimental.pallas import tpu as pltpu
from jax.experimental.pallas import tpu_sc as plsc

B = 1024
K = 100000
D = 128
KT = 1024
KP = ((K + KT - 1) // KT) * KT
C = 128
S = 20
BT = 32


def _nrm(x):
    ss = jnp.sum(x * x, axis=1, keepdims=True)
    return x * lax.rsqrt(jnp.maximum(ss, jnp.float32(1e-24)))


SEG = 128
NSEG = KP // SEG


def _scores_body(state_ref, mem_ref, out_ref, segmax_ref, qn_ref):
    k = pl.program_id(0)

    @pl.when(k == 0)
    def _():
        qn_ref[...] = _nrm(state_ref[...]).astype(jnp.bfloat16)

    m = mem_ref[...]
    s = lax.dot_general(qn_ref[...], _nrm(m).astype(jnp.bfloat16),
                        (((1,), (1,)), ((), ())),
                        preferred_element_type=jnp.float32)
    col = k * KT + lax.broadcasted_iota(jnp.int32, (B, KT), 1)
    s = jnp.where(col >= K, jnp.float32(-2.0), s)
    out_ref[...] = s
    segmax_ref[...] = jnp.max(s.reshape(B, KT // SEG, SEG), axis=2)[None]


def _scores(state_past, memory_past):
    mem_p = jnp.pad(memory_past, ((0, KP - K), (0, 0)))
    return pl.pallas_call(
        _scores_body,
        grid=(KP // KT,),
        in_specs=[
            pl.BlockSpec((B, D), lambda k: (0, 0)),
            pl.BlockSpec((KT, D), lambda k: (k, 0)),
        ],
        out_specs=[
            pl.BlockSpec((B, KT), lambda k: (0, k)),
            pl.BlockSpec((1, B, KT // SEG), lambda k: (k, 0, 0)),
        ],
        out_shape=[
            jax.ShapeDtypeStruct((B, KP), jnp.float32),
            jax.ShapeDtypeStruct((KP // KT, B, KT // SEG), jnp.float32),
        ],
        scratch_shapes=[pltpu.VMEM((B, D), jnp.bfloat16)],
    )(state_past, mem_p)


def _bdot(x, w):
    return lax.dot_general(x.astype(jnp.bfloat16), w.astype(jnp.bfloat16),
                           (((1,), (0,)), ((), ())),
                           preferred_element_type=jnp.float32)


def _selector_body(state_ref, mps_ref,
                   wq1, bq1, wq2, bq2, wq3, bq3,
                   wm1, bm1, wm2, bm2, wm3, bm3,
                   q_ref, mo_ref):
    x = state_ref[...]
    h = jnp.maximum(_bdot(x, wq1[...]) + bq1[...], 0.0)
    h = jnp.maximum(_bdot(h, wq2[...]) + bq2[...], 0.0)
    q_ref[...] = _bdot(h, wq3[...]) + bq3[...]

    m = mps_ref[...].reshape(BT * C, D)
    hm = jnp.maximum(_bdot(m, wm1[...]) + bm1[...], 0.0)
    hm = jnp.maximum(_bdot(hm, wm2[...]) + bm2[...], 0.0)
    mo_ref[...] = (_bdot(hm, wm3[...]) + bm3[...]).reshape(BT, C, D)


def _selector(state_past, mem_past_sel,
              Wq1, bq1, Wq2, bq2, Wq3, bq3,
              Wm1, bm1, Wm2, bm2, Wm3, bm3):
    w_specs = []
    w_args = []
    for w, b in ((Wq1, bq1), (Wq2, bq2), (Wq3, bq3),
                 (Wm1, bm1), (Wm2, bm2), (Wm3, bm3)):
        b2 = b.reshape(1, -1)
        w_specs.append(pl.BlockSpec(w.shape, lambda i: (0, 0)))
        w_specs.append(pl.BlockSpec(b2.shape, lambda i: (0, 0)))
        w_args.append(w)
        w_args.append(b2)
    return pl.pallas_call(
        _selector_body,
        grid=(B // BT,),
        in_specs=[
            pl.BlockSpec((BT, D), lambda i: (i, 0)),
            pl.BlockSpec((BT, C, D), lambda i: (i, 0, 0)),
        ] + w_specs,
        out_specs=[
            pl.BlockSpec((BT, D), lambda i: (i, 0)),
            pl.BlockSpec((BT, C, D), lambda i: (i, 0, 0)),
        ],
        out_shape=[
            jax.ShapeDtypeStruct((B, D), jnp.float32),
            jax.ShapeDtypeStruct((B, C, D), jnp.float32),
        ],
    )(state_past, mem_past_sel, *w_args)


def _l2n(x, axis):
    n = jnp.linalg.norm(x, axis=axis, keepdims=True)
    return x / jnp.clip(n, 1e-12, None)


def _sc_gather_fut(memory_fut, idx_flat):
    info = plsc.get_sparse_core_info()
    nw = info.num_cores * info.num_subcores
    n = idx_flat.shape[0]
    per_w = n // nw
    mesh = plsc.VectorSubcoreMesh(core_axis_name="c", subcore_axis_name="s")

    @functools.partial(
        pl.kernel, mesh=mesh,
        out_type=jax.ShapeDtypeStruct((n, D), jnp.float32),
        scratch_types=[
            pltpu.VMEM((per_w,), jnp.int32),
            pltpu.VMEM((per_w, D), jnp.float32),
            pltpu.SemaphoreType.DMA,
        ],
    )
    def k(table_hbm, idx_hbm, out_hbm, idx_v, rows_v, sem):
        wid = lax.axis_index("s") * info.num_cores + lax.axis_index("c")
        base = wid * per_w
        pltpu.sync_copy(idx_hbm.at[pl.ds(base, per_w)], idx_v)
        pltpu.async_copy(table_hbm.at[idx_v], rows_v, sem).wait()
        pltpu.sync_copy(rows_v, out_hbm.at[pl.ds(base, per_w)])

    return k(memory_fut, idx_flat)


def kernel(state_past, memory_past, memory_fut, Wq1, bq1, Wq2, bq2, Wq3, bq3,
           Wm1, bm1, Wm2, bm2, Wm3, bm3):
    scores, segmax3 = _scores(state_past, memory_past)
    segmax = jnp.transpose(segmax3, (1, 0, 2)).reshape(B, NSEG)
    _, seg1 = jax.lax.top_k(segmax, C)
    g1 = jnp.sort(seg1, axis=1)
    s1 = jnp.take_along_axis(scores.reshape(B, NSEG, SEG), g1[:, :, None],
                             axis=1)
    s1f = s1.reshape(B, C * SEG)
    m2 = jnp.max(s1f.reshape(B, (C * SEG) // 16, 16), axis=2)
    _, ch2 = jax.lax.top_k(m2, C)
    g2 = jnp.sort(ch2, axis=1)
    s2 = jnp.take_along_axis(s1f.reshape(B, (C * SEG) // 16, 16),
                             g2[:, :, None], axis=1)
    _, p2 = jax.lax.top_k(s2.reshape(B, C * 16), C)
    pos1 = jnp.take_along_axis(g2, p2 // 16, axis=1) * 16 + p2 % 16
    index_max = (jnp.take_along_axis(g1, pos1 // SEG, axis=1) * SEG
                 + pos1 % SEG)
    mem_past_sel = jnp.take(memory_past, index_max, axis=0)
    q, mo = _selector(state_past, mem_past_sel,
                      Wq1, bq1, Wq2, bq2, Wq3, bq3,
                      Wm1, bm1, Wm2, bm2, Wm3, bm3)
    w2 = jnp.einsum('bqd,bcd->bc', _l2n(q[:, None, :], 2), _l2n(mo, 2))
    _, smi = jax.lax.top_k(w2, S)
    idx_final = jnp.take_along_axis(index_max, smi, axis=1)
    return _sc_gather_fut(memory_fut, idx_final.reshape(-1)).reshape(B, S, D)
